# Initial kernel scaffold; baseline (speedup 1.0000x reference)
import jax
import jax.numpy as jnp
from jax.experimental import pallas as pl


def kernel(idx, weight):
    raise NotImplementedError("write your pallas kernel here")


# speedup vs baseline: 1.5764x; 1.5764x over previous
import functools

import jax
import jax.numpy as jnp
from jax import lax
from jax.experimental import pallas as pl
from jax.experimental.pallas import tpu as pltpu
from jax.experimental.pallas import tpu_sc as plsc

VOCAB = 1000000
D = 32
B = 16384 * 26
NC, NS = 2, 16
NW = NC * NS
BPW = B // NW
CH = 1664
NCH = BPW // CH

_mesh = plsc.VectorSubcoreMesh(core_axis_name="c", subcore_axis_name="s")


@functools.partial(
    pl.kernel,
    out_type=jax.ShapeDtypeStruct((B, D), jnp.float32),
    mesh=_mesh,
    scratch_types=[
        pltpu.VMEM((BPW,), jnp.int32),
        pltpu.VMEM((CH, D), jnp.float32),
        pltpu.VMEM((CH, D), jnp.float32),
        pltpu.SemaphoreType.DMA,
        pltpu.SemaphoreType.DMA,
        pltpu.SemaphoreType.DMA,
        pltpu.SemaphoreType.DMA,
    ],
    compiler_params=pltpu.CompilerParams(use_tc_tiling_on_sc=False),
)
def _embed_sc(idx_hbm, w_hbm, out_hbm, idx_v, buf0, buf1, g0, g1, o0, o1):
    wid = lax.axis_index("s") * NC + lax.axis_index("c")
    base = wid * BPW
    pltpu.sync_copy(idx_hbm.at[pl.ds(base, BPW)], idx_v)

    bufs = (buf0, buf1)
    gsems = (g0, g1)
    osems = (o0, o1)

    gather = [None] * NCH
    outcp = [None] * NCH
    gather[0] = pltpu.async_copy(
        w_hbm.at[idx_v.at[pl.ds(0, CH)]], bufs[0], gsems[0])
    for c in range(NCH):
        n = c + 1
        if n < NCH:
            if n >= 2:
                outcp[n - 2].wait()
            gather[n] = pltpu.async_copy(
                w_hbm.at[idx_v.at[pl.ds(n * CH, CH)]], bufs[n % 2], gsems[n % 2])
        gather[c].wait()
        outcp[c] = pltpu.async_copy(
            bufs[c % 2], out_hbm.at[pl.ds(base + c * CH, CH)], osems[c % 2])
    outcp[NCH - 2].wait()
    outcp[NCH - 1].wait()


def kernel(idx, weight):
    flat = idx.reshape(-1).astype(jnp.int32)
    out = _embed_sc(flat, weight)
    return out.reshape(idx.shape + (D,))
